# Initial kernel scaffold; baseline (speedup 1.0000x reference)
import jax
import jax.numpy as jnp
from jax.experimental import pallas as pl


def kernel(X, edge_index, edge_vals, W, b):
    raise NotImplementedError("write your pallas kernel here")


# speedup vs baseline: 4.4604x; 4.4604x over previous
import functools

import jax
import jax.numpy as jnp
from jax import lax
from jax.experimental import pallas as pl
from jax.experimental.pallas import tpu as pltpu
from jax.experimental.pallas import tpu_sc as plsc

NC = 2
NS = 16
NW = NC * NS
L = 16


def _mm_body(x_ref, wt_ref, b_ref, h_ref):
    h_ref[...] = (
        jnp.dot(x_ref[...], wt_ref[...], preferred_element_type=jnp.float32)
        + b_ref[...]
    )


@functools.partial(jax.jit, static_argnames=())
def _matmul(x, wt, b2d):
    n, d_in = x.shape
    d_out = wt.shape[1]
    blk = 1000 if n % 1000 == 0 else n
    grid = n // blk
    return pl.pallas_call(
        _mm_body,
        grid=(grid,),
        in_specs=[
            pl.BlockSpec((blk, d_in), lambda i: (i, 0)),
            pl.BlockSpec((d_in, d_out), lambda i: (0, 0)),
            pl.BlockSpec((1, d_out), lambda i: (0, 0)),
        ],
        out_specs=pl.BlockSpec((blk, d_out), lambda i: (i, 0)),
        out_shape=jax.ShapeDtypeStruct((n, d_out), jnp.float32),
    )(x, wt, b2d)


def _comb_body(p_ref, o_ref):
    o_ref[...] = jnp.maximum(p_ref[0] + p_ref[1], 0.0)


def _combine(partials, n):
    _, _, d = partials.shape
    blk = 1000 if n % 1000 == 0 else n
    grid = n // blk
    return pl.pallas_call(
        _comb_body,
        grid=(grid,),
        in_specs=[pl.BlockSpec((2, blk, d), lambda i: (0, i, 0))],
        out_specs=pl.BlockSpec((blk, d), lambda i: (i, 0)),
        out_shape=jax.ShapeDtypeStruct((n, d), jnp.float32),
    )(partials)


def _sc_mp(h, rows, cols, vals):
    n, d = h.shape
    e = rows.shape[0]
    assert e % NW == 0
    e_tile = e // NW
    k = max(kk for kk in range(8, 129, 8) if e_tile % kk == 0)
    n_chunks = e_tile // k
    dk = 128
    n_pad = -(-n // (NS * dk)) * (NS * dk)
    rows_tile = n_pad // NS
    n_drain = rows_tile // dk

    mesh = plsc.VectorSubcoreMesh(
        core_axis_name="c", subcore_axis_name="s",
        num_cores=NC, num_subcores=NS)

    @functools.partial(
        pl.kernel,
        out_type=jax.ShapeDtypeStruct((NC, n_pad, d), jnp.float32),
        mesh=mesh,
        scratch_types=[
            pltpu.VMEM((k,), jnp.int32),
            pltpu.VMEM((k,), jnp.int32),
            pltpu.VMEM((k,), jnp.float32),
            pltpu.VMEM((k, d), jnp.float32),
            pltpu.VMEM((dk, d), jnp.float32),
            pltpu.VMEM_SHARED((n_pad, d), jnp.float32),
            pltpu.SemaphoreType.DMA,
        ],
    )
    def mp(h_hbm, rows_hbm, cols_hbm, vals_hbm, out_hbm,
           rows_v, cols_v, vals_v, msg_v, drain_v, acc_sh, sem):
        c = lax.axis_index("c")
        s = lax.axis_index("s")
        wid = c * NS + s

        def zrow(r, _):
            for j in range(d // L):
                drain_v[r, pl.ds(j * L, L)] = jnp.zeros((L,), jnp.float32)
            return 0

        lax.fori_loop(0, dk, zrow, 0)

        def zchunk(i, _):
            pltpu.sync_copy(drain_v, acc_sh.at[pl.ds(s * rows_tile + i * dk, dk)])
            return 0

        lax.fori_loop(0, n_drain, zchunk, 0)
        plsc.subcore_barrier()

        base0 = wid * e_tile

        def chunk(ci, _):
            base = base0 + ci * k
            pltpu.sync_copy(rows_hbm.at[pl.ds(base, k)], rows_v)
            pltpu.sync_copy(cols_hbm.at[pl.ds(base, k)], cols_v)
            pltpu.sync_copy(vals_hbm.at[pl.ds(base, k)], vals_v)
            pltpu.async_copy(h_hbm.at[cols_v], msg_v, sem).wait()

            def scale(g, _):
                vv = vals_v[pl.ds(g * L, L)]
                for l in range(L):
                    sv = jnp.full((L,), vv[l], jnp.float32)
                    e0 = g * L + l
                    for j in range(d // L):
                        sl = pl.ds(j * L, L)
                        msg_v[e0, sl] = msg_v[e0, sl] * sv
                return 0

            lax.fori_loop(0, k // L, scale, 0)
            pltpu.sync_copy(msg_v, acc_sh.at[rows_v], add=True)
            return 0

        lax.fori_loop(0, n_chunks, chunk, 0)
        plsc.subcore_barrier()

        def drain(i, _):
            r0 = s * rows_tile + i * dk
            pltpu.sync_copy(acc_sh.at[pl.ds(r0, dk)], drain_v)
            pltpu.sync_copy(drain_v, out_hbm.at[c, pl.ds(r0, dk)])
            return 0

        lax.fori_loop(0, n_drain, drain, 0)

    return mp(h, rows, cols, vals)


def kernel(X, edge_index, edge_vals, W, b):
    h = _matmul(X, W.T, b.reshape(1, -1))
    rows = edge_index[0]
    cols = edge_index[1]
    partials = _sc_mp(h, rows, cols, edge_vals)
    return _combine(partials, X.shape[0])


# speedup vs baseline: 8.1768x; 1.8332x over previous
import functools

import jax
import jax.numpy as jnp
from jax import lax
from jax.experimental import pallas as pl
from jax.experimental.pallas import tpu as pltpu
from jax.experimental.pallas import tpu_sc as plsc

NC = 2
NS = 16
NW = NC * NS
L = 16


def _mm_body(x_ref, wt_ref, b_ref, h_ref):
    h_ref[...] = (
        jnp.dot(x_ref[...], wt_ref[...], preferred_element_type=jnp.float32)
        + b_ref[...]
    )


@functools.partial(jax.jit, static_argnames=())
def _matmul(x, wt, b2d):
    n, d_in = x.shape
    d_out = wt.shape[1]
    blk = 1000 if n % 1000 == 0 else n
    grid = n // blk
    return pl.pallas_call(
        _mm_body,
        grid=(grid,),
        in_specs=[
            pl.BlockSpec((blk, d_in), lambda i: (i, 0)),
            pl.BlockSpec((d_in, d_out), lambda i: (0, 0)),
            pl.BlockSpec((1, d_out), lambda i: (0, 0)),
        ],
        out_specs=pl.BlockSpec((blk, d_out), lambda i: (i, 0)),
        out_shape=jax.ShapeDtypeStruct((n, d_out), jnp.float32),
    )(x, wt, b2d)


def _comb_body(p_ref, o_ref):
    o_ref[...] = jnp.maximum(p_ref[0] + p_ref[1], 0.0)


def _combine(partials, n):
    _, _, d = partials.shape
    blk = 1000 if n % 1000 == 0 else n
    grid = n // blk
    return pl.pallas_call(
        _comb_body,
        grid=(grid,),
        in_specs=[pl.BlockSpec((2, blk, d), lambda i: (0, i, 0))],
        out_specs=pl.BlockSpec((blk, d), lambda i: (i, 0)),
        out_shape=jax.ShapeDtypeStruct((n, d), jnp.float32),
    )(partials)


def _sc_mp(h, packed, e, k, n_chunks):
    n, d = h.shape
    dk = 128
    n_pad = -(-n // (NS * dk)) * (NS * dk)
    rows_tile = n_pad // NS
    n_drain = rows_tile // dk

    mesh = plsc.VectorSubcoreMesh(
        core_axis_name="c", subcore_axis_name="s",
        num_cores=NC, num_subcores=NS)

    @functools.partial(
        pl.kernel,
        out_type=jax.ShapeDtypeStruct((NC, n_pad, d), jnp.float32),
        mesh=mesh,
        scratch_types=[
            pltpu.VMEM((3, k), jnp.int32),
            pltpu.VMEM((3, k), jnp.int32),
            pltpu.VMEM((k, d), jnp.float32),
            pltpu.VMEM((k, d), jnp.float32),
            pltpu.VMEM((dk, d), jnp.float32),
            pltpu.VMEM_SHARED((n_pad, d), jnp.float32),
            pltpu.SemaphoreType.DMA,
            pltpu.SemaphoreType.DMA,
            pltpu.SemaphoreType.DMA,
            pltpu.SemaphoreType.DMA,
        ],
    )
    def mp(h_hbm, pk_hbm, out_hbm,
           eb_a, eb_b, msg_a, msg_b, drain_v, acc_sh,
           sem_ia, sem_ib, sem_ga, sem_gb):
        c = lax.axis_index("c")
        s = lax.axis_index("s")
        wid = c * NS + s

        def zrow(r, _):
            for j in range(d // L):
                drain_v[r, pl.ds(j * L, L)] = jnp.zeros((L,), jnp.float32)
            return 0

        lax.fori_loop(0, dk, zrow, 0)

        def zchunk(i, _):
            pltpu.sync_copy(drain_v, acc_sh.at[pl.ds(s * rows_tile + i * dk, dk)])
            return 0

        lax.fori_loop(0, n_drain, zchunk, 0)
        plsc.subcore_barrier()

        chunk0 = wid * n_chunks

        def load_idx(ci, eb, sem):
            pltpu.async_copy(pk_hbm.at[chunk0 + ci], eb, sem)

        def wait_idx(eb, sem):
            pltpu.make_async_copy(pk_hbm.at[0], eb, sem).wait()

        def start_gather(eb, msg, sem):
            pltpu.async_copy(h_hbm.at[eb.at[1]], msg, sem)

        def wait_gather(msg, sem):
            pltpu.make_async_copy(h_hbm.at[pl.ds(0, k)], msg, sem).wait()

        def scale(eb, msg):
            def grp(g, _):
                vv = lax.bitcast_convert_type(
                    eb[2, pl.ds(g * L, L)], jnp.float32)
                for l in range(L):
                    sv = jnp.full((L,), vv[l], jnp.float32)
                    e0 = g * L + l
                    for j in range(d // L):
                        sl = pl.ds(j * L, L)
                        msg[e0, sl] = msg[e0, sl] * sv
                return 0

            lax.fori_loop(0, k // L, grp, 0)

        def scatter(eb, msg):
            pltpu.sync_copy(msg, acc_sh.at[eb.at[0]], add=True)

        load_idx(0, eb_a, sem_ia)
        wait_idx(eb_a, sem_ia)
        start_gather(eb_a, msg_a, sem_ga)
        wait_gather(msg_a, sem_ga)
        scale(eb_a, msg_a)
        scatter(eb_a, msg_a)
        n_pairs = (n_chunks - 1) // 2
        leftover = (n_chunks - 1) - 2 * n_pairs
        if n_pairs > 0:
            load_idx(1, eb_a, sem_ia)
            wait_idx(eb_a, sem_ia)
            start_gather(eb_a, msg_a, sem_ga)
            load_idx(2, eb_b, sem_ib)
            wait_idx(eb_b, sem_ib)

            def pair(i, _):
                ca = 2 * i + 1
                start_gather(eb_b, msg_b, sem_gb)
                wait_gather(msg_a, sem_ga)
                scale(eb_a, msg_a)
                scatter(eb_a, msg_a)

                @pl.when(ca + 2 < 2 * n_pairs + 1)
                def _():
                    load_idx(ca + 2, eb_a, sem_ia)
                    wait_idx(eb_a, sem_ia)
                    start_gather(eb_a, msg_a, sem_ga)

                wait_gather(msg_b, sem_gb)
                scale(eb_b, msg_b)
                scatter(eb_b, msg_b)

                @pl.when(ca + 3 < 2 * n_pairs + 2)
                def _():
                    load_idx(ca + 3, eb_b, sem_ib)
                    wait_idx(eb_b, sem_ib)

                return 0

            lax.fori_loop(0, n_pairs, pair, 0)
        if leftover:
            ci = n_chunks - 1
            load_idx(ci, eb_a, sem_ia)
            wait_idx(eb_a, sem_ia)
            start_gather(eb_a, msg_a, sem_ga)
            wait_gather(msg_a, sem_ga)
            scale(eb_a, msg_a)
            scatter(eb_a, msg_a)

        plsc.subcore_barrier()

        def drain(i, _):
            r0 = s * rows_tile + i * dk
            pltpu.sync_copy(acc_sh.at[pl.ds(r0, dk)], drain_v)
            pltpu.sync_copy(drain_v, out_hbm.at[c, pl.ds(r0, dk)])
            return 0

        lax.fori_loop(0, n_drain, drain, 0)

    return mp(h, packed)


def kernel(X, edge_index, edge_vals, W, b):
    h = _matmul(X, W.T, b.reshape(1, -1))
    rows = edge_index[0]
    cols = edge_index[1]
    e = rows.shape[0]
    e_tile = e // NW
    k = max(kk for kk in range(8, 129, 8) if e_tile % kk == 0)
    n_chunks = e_tile // k
    packed = jnp.stack(
        [rows.reshape(-1, k), cols.reshape(-1, k),
         lax.bitcast_convert_type(edge_vals, jnp.int32).reshape(-1, k)],
        axis=1)
    partials = _sc_mp(h, packed, e, k, n_chunks)
    return _combine(partials, X.shape[0])
